# baseline (speedup 1.0000x reference)
import jax
import jax.numpy as jnp
from jax import lax
from jax.experimental import pallas as pl
from jax.experimental.pallas import tpu as pltpu
from jax.experimental.pallas import tpu_sc as plsc

_K = 32
_L = 16
_B = 128


def _dot_body(idx_hbm, u_hbm, p_hbm, out_hbm,
              idx_v, u_blk, p_blk, out_v, sem):
    is_lead = lax.axis_index("s") == 0

    @pl.when(is_lead)
    def _():
        pltpu.sync_copy(idx_hbm, idx_v)
        iv = idx_v[...]
        iu = iv[0]
        ip = iv[1]
        bu = pl.multiple_of((iu // _B) * _B, _B)
        bp = pl.multiple_of((ip // _B) * _B, _B)
        cu = pltpu.async_copy(u_hbm.at[:, pl.ds(bu, _B)], u_blk, sem)
        cp = pltpu.async_copy(p_hbm.at[:, pl.ds(bp, _B)], p_blk, sem)
        cu.wait()
        cp.wait()
        lanes = lax.iota(jnp.int32, _L)
        cu_idx = jnp.broadcast_to(iu % _B, (_L,))
        cp_idx = jnp.broadcast_to(ip % _B, (_L,))
        u0 = plsc.load_gather(u_blk, [lanes, cu_idx])
        u1 = plsc.load_gather(u_blk, [lanes + _L, cu_idx])
        p0 = plsc.load_gather(p_blk, [lanes, cp_idx])
        p1 = plsc.load_gather(p_blk, [lanes + _L, cp_idx])
        prod = u0 * p0 + u1 * p1
        out_v[...] = jnp.zeros((_L,), jnp.float32)
        plsc.addupdate_scatter(out_v, [jnp.zeros((_L,), jnp.int32)], prod)
        pltpu.sync_copy(out_v, out_hbm)


_sc_dot = pl.kernel(
    _dot_body,
    out_type=jax.ShapeDtypeStruct((_L,), jnp.float32),
    mesh=plsc.VectorSubcoreMesh(
        core_axis_name="c", subcore_axis_name="s", num_cores=1,
        num_subcores=1),
    compiler_params=pltpu.CompilerParams(
        needs_layout_passes=False, use_tc_tiling_on_sc=True,
        skip_device_barrier=True, disable_bounds_checks=True,
        disable_semaphore_checks=True),
    scratch_types=[
        pltpu.VMEM((_L,), jnp.int32),
        pltpu.VMEM((_K, _B), jnp.float32),
        pltpu.VMEM((_K, _B), jnp.float32),
        pltpu.VMEM((_L,), jnp.float32),
        pltpu.SemaphoreType.DMA,
    ],
)


def kernel(i_user, i_product, U, P):
    idx = jnp.zeros((_L,), jnp.int32)
    idx = idx.at[0].set(i_user).at[1].set(i_product)
    out = _sc_dot(idx, U.T, P.T)
    return out[0]


# speedup vs baseline: 1.0008x; 1.0008x over previous
import jax
import jax.numpy as jnp
from jax import lax
from jax.experimental import pallas as pl
from jax.experimental.pallas import tpu as pltpu
from jax.experimental.pallas import tpu_sc as plsc

_K = 32
_L = 16
_B = 128


def _dot_body(idx_hbm, u_hbm, p_hbm, out_hbm,
              idx_v, u_blk, p_blk, out_v, sem):
    is_lead = lax.axis_index("s") == 0

    @pl.when(is_lead)
    def _():
        out_v[...] = jnp.zeros((_L,), jnp.float32)
        pltpu.sync_copy(idx_hbm, idx_v)
        iv = idx_v[...]
        iu = iv[0]
        ip = iv[1]
        bu = pl.multiple_of((iu // _B) * _B, _B)
        bp = pl.multiple_of((ip // _B) * _B, _B)
        cu = pltpu.async_copy(u_hbm.at[:, pl.ds(bu, _B)], u_blk, sem)
        cp = pltpu.async_copy(p_hbm.at[:, pl.ds(bp, _B)], p_blk, sem)
        cu.wait()
        cp.wait()
        lanes = lax.iota(jnp.int32, _L)
        cu_idx = jnp.broadcast_to(iu % _B, (_L,))
        cp_idx = jnp.broadcast_to(ip % _B, (_L,))
        u0 = plsc.load_gather(u_blk, [lanes, cu_idx])
        u1 = plsc.load_gather(u_blk, [lanes + _L, cu_idx])
        p0 = plsc.load_gather(p_blk, [lanes, cp_idx])
        p1 = plsc.load_gather(p_blk, [lanes + _L, cp_idx])
        prod = u0 * p0 + u1 * p1
        plsc.addupdate_scatter(out_v, [jnp.zeros((_L,), jnp.int32)], prod)
        pltpu.sync_copy(out_v, out_hbm)


_sc_dot = pl.kernel(
    _dot_body,
    out_type=jax.ShapeDtypeStruct((_L,), jnp.float32),
    mesh=plsc.VectorSubcoreMesh(
        core_axis_name="c", subcore_axis_name="s", num_cores=1,
        num_subcores=1),
    compiler_params=pltpu.CompilerParams(
        needs_layout_passes=False, use_tc_tiling_on_sc=True,
        skip_device_barrier=True, disable_bounds_checks=True,
        disable_semaphore_checks=True),
    scratch_types=[
        pltpu.VMEM((_L,), jnp.int32),
        pltpu.VMEM((_K, _B), jnp.float32),
        pltpu.VMEM((_K, _B), jnp.float32),
        pltpu.VMEM((_L,), jnp.float32),
        pltpu.SemaphoreType.DMA,
    ],
)


def kernel(i_user, i_product, U, P):
    idx = jnp.zeros((_L,), jnp.int32)
    idx = idx.at[0].set(i_user).at[1].set(i_product)
    out = _sc_dot(idx, U.T, P.T)
    return out[0]


# speedup vs baseline: 1.0033x; 1.0026x over previous
import jax
import jax.numpy as jnp
from jax import lax
from jax.experimental import pallas as pl
from jax.experimental.pallas import tpu as pltpu
from jax.experimental.pallas import tpu_sc as plsc

_K = 32
_L = 16
_B = 128


def _dot_body(iu_hbm, ip_hbm, u_hbm, p_hbm, out_hbm,
              idx_v, u_blk, p_blk, out_v, sem):
    is_lead = lax.axis_index("s") == 0

    @pl.when(is_lead)
    def _():
        out_v[...] = jnp.zeros((_L,), jnp.float32)
        ci = pltpu.async_copy(iu_hbm, idx_v.at[pl.ds(0, 1)], sem)
        cj = pltpu.async_copy(ip_hbm, idx_v.at[pl.ds(8, 1)], sem)
        ci.wait()
        cj.wait()
        iv = idx_v[...]
        iu = iv[0]
        ip = iv[8]
        bu = pl.multiple_of((iu // _B) * _B, _B)
        bp = pl.multiple_of((ip // _B) * _B, _B)
        cu = pltpu.async_copy(u_hbm.at[:, pl.ds(bu, _B)], u_blk, sem)
        cp = pltpu.async_copy(p_hbm.at[:, pl.ds(bp, _B)], p_blk, sem)
        cu.wait()
        cp.wait()
        lanes = lax.iota(jnp.int32, _L)
        cu_idx = jnp.broadcast_to(iu % _B, (_L,))
        cp_idx = jnp.broadcast_to(ip % _B, (_L,))
        u0 = plsc.load_gather(u_blk, [lanes, cu_idx])
        u1 = plsc.load_gather(u_blk, [lanes + _L, cu_idx])
        p0 = plsc.load_gather(p_blk, [lanes, cp_idx])
        p1 = plsc.load_gather(p_blk, [lanes + _L, cp_idx])
        prod = u0 * p0 + u1 * p1
        plsc.addupdate_scatter(out_v, [jnp.zeros((_L,), jnp.int32)], prod)
        pltpu.sync_copy(out_v, out_hbm)


_sc_dot = pl.kernel(
    _dot_body,
    out_type=jax.ShapeDtypeStruct((_L,), jnp.float32),
    mesh=plsc.VectorSubcoreMesh(
        core_axis_name="c", subcore_axis_name="s", num_cores=1,
        num_subcores=1),
    compiler_params=pltpu.CompilerParams(
        needs_layout_passes=False, use_tc_tiling_on_sc=True,
        skip_device_barrier=True, disable_bounds_checks=True,
        disable_semaphore_checks=True),
    scratch_types=[
        pltpu.VMEM((_L,), jnp.int32),
        pltpu.VMEM((_K, _B), jnp.float32),
        pltpu.VMEM((_K, _B), jnp.float32),
        pltpu.VMEM((_L,), jnp.float32),
        pltpu.SemaphoreType.DMA,
    ],
)


def kernel(i_user, i_product, U, P):
    iu = jnp.reshape(jnp.asarray(i_user, jnp.int32), (1,))
    ip = jnp.reshape(jnp.asarray(i_product, jnp.int32), (1,))
    out = _sc_dot(iu, ip, U.T, P.T)
    return out[0]
